# initial kernel scaffold (unmeasured)
import jax
import jax.numpy as jnp
from jax import lax
from jax.experimental import pallas as pl
from jax.experimental.pallas import tpu as pltpu

N_DEV = 32
BLK = 128


def kernel(x, w_mat):
    m_full, k_shard = x.shape
    k_full, n = w_mat.shape
    m_out = m_full // N_DEV

    def body(x_ref, w_ref, out_ref, xfull_ref, send_sems, recv_sems):
        my = lax.axis_index("i")

        xfull_ref[:, pl.ds(my * BLK, BLK)] = x_ref[pl.ds(my * BLK, BLK), :]

        rdmas = []
        for s in range(1, N_DEV):
            dst = lax.rem(my + s, N_DEV)
            rdma = pltpu.make_async_remote_copy(
                src_ref=x_ref.at[pl.ds(dst * BLK, BLK), :],
                dst_ref=xfull_ref.at[:, pl.ds(my * BLK, BLK)],
                send_sem=send_sems.at[s],
                recv_sem=recv_sems.at[s],
                device_id=(dst,),
                device_id_type=pl.DeviceIdType.MESH,
            )
            rdma.start()
            rdmas.append(rdma)

        for rdma in rdmas:
            rdma.wait()

        acc = jnp.dot(
            xfull_ref[:, :], w_ref[:, :], preferred_element_type=jnp.float32
        )
        out_ref[:, :] = acc * (1.0 / (1.0 + jnp.exp(-acc)))

    return pl.pallas_call(
        body,
        out_shape=jax.ShapeDtypeStruct((m_out, n), jnp.float32),
        in_specs=[
            pl.BlockSpec(memory_space=pltpu.VMEM),
            pl.BlockSpec(memory_space=pltpu.VMEM),
        ],
        out_specs=pl.BlockSpec(memory_space=pltpu.VMEM),
        scratch_shapes=[
            pltpu.VMEM((m_out, k_full), jnp.float32),
            pltpu.SemaphoreType.DMA((N_DEV,)),
            pltpu.SemaphoreType.DMA((N_DEV,)),
        ],
    )(x, w_mat)


# baseline (device time: 53668 ns/iter reference)
import jax
import jax.numpy as jnp
from jax import lax
from jax.experimental import pallas as pl
from jax.experimental.pallas import tpu as pltpu

N_DEV = 32
BLK = 128


def kernel(x, w_mat):
    m_full, k_shard = x.shape
    k_full, n = w_mat.shape
    m_out = m_full // N_DEV

    def body(x_ref, w_ref, out_ref, xfull_ref, send_sems, recv_sems):
        my = lax.axis_index("i")

        xfull_ref[:, pl.ds(my * BLK, BLK)] = x_ref[pl.ds(my * BLK, BLK), :]

        rdmas = []
        for s in range(1, N_DEV):
            dst = lax.rem(my + s, N_DEV)
            rdma = pltpu.make_async_remote_copy(
                src_ref=x_ref.at[pl.ds(dst * BLK, BLK), :],
                dst_ref=xfull_ref.at[:, pl.ds(my * BLK, BLK)],
                send_sem=send_sems.at[s],
                recv_sem=recv_sems.at[s],
                device_id=(dst,),
                device_id_type=pl.DeviceIdType.MESH,
            )
            rdma.start()
            rdmas.append(rdma)

        for rdma in rdmas:
            rdma.wait()

        acc = jnp.dot(
            xfull_ref[:, :], w_ref[:, :], preferred_element_type=jnp.float32
        )
        out_ref[:, :] = acc * (1.0 / (1.0 + jnp.exp(-acc)))

    return pl.pallas_call(
        body,
        out_shape=jax.ShapeDtypeStruct((m_out, n), jnp.float32),
        in_specs=[
            pl.BlockSpec(memory_space=pltpu.VMEM),
            pl.BlockSpec(memory_space=pltpu.VMEM),
        ],
        out_specs=pl.BlockSpec(memory_space=pltpu.VMEM),
        scratch_shapes=[
            pltpu.VMEM((m_out, k_full), jnp.float32),
            pltpu.SemaphoreType.DMA((N_DEV,)),
            pltpu.SemaphoreType.DMA((N_DEV,)),
        ],
        compiler_params=pltpu.CompilerParams(
            vmem_limit_bytes=64 * 1024 * 1024,
        ),
    )(x, w_mat)
